# baseline (speedup 1.0000x reference)
import functools

import jax
import jax.numpy as jnp
from jax import lax
from jax.experimental import pallas as pl
from jax.experimental.pallas import tpu as pltpu
from jax.experimental.pallas import tpu_sc as plsc

NUM_ROWS = 3
DIM = 256

NUM_CORES = 2
NUM_SUBCORES = 16
NUM_WORKERS = NUM_CORES * NUM_SUBCORES
LANES = 16

N_TOTAL = 16 * 128 * 128
RPW = N_TOTAL // NUM_WORKERS
CHUNK = 128
NCHUNKS = RPW // CHUNK


@functools.partial(
    pl.kernel,
    out_type=jax.ShapeDtypeStruct((N_TOTAL, DIM), jnp.float32),
    mesh=plsc.VectorSubcoreMesh(core_axis_name="c", subcore_axis_name="s"),
    scratch_types=[
        pltpu.VMEM((RPW,), jnp.int32),
        pltpu.VMEM((CHUNK, DIM), jnp.float32),
        pltpu.SemaphoreType.DMA,
    ],
)
def _sc_expand(idx_hbm, table_hbm, out_hbm, idx_v, rows_v, gsem):
    wid = lax.axis_index("s") * NUM_CORES + lax.axis_index("c")
    base = wid * RPW
    pltpu.sync_copy(idx_hbm.at[pl.ds(base, RPW)], idx_v)

    def bucket_body(i, carry):
        s = idx_v[pl.ds(i * LANES, LANES)]
        idx_v[pl.ds(i * LANES, LANES)] = jnp.clip(s - 1, 0, NUM_ROWS - 1)
        return carry

    lax.fori_loop(0, RPW // LANES, bucket_body, 0)

    def chunk_body(c, carry):
        idx_slice = idx_v.at[pl.ds(c * CHUNK, CHUNK)]
        pltpu.async_copy(table_hbm.at[idx_slice], rows_v, gsem).wait()
        pltpu.sync_copy(rows_v, out_hbm.at[pl.ds(base + c * CHUNK, CHUNK)])
        return carry

    lax.fori_loop(0, NCHUNKS, chunk_body, 0)


def kernel(path_length, bucket_embedding):
    shape = path_length.shape
    flat_idx = path_length.reshape(-1).astype(jnp.int32)
    out = _sc_expand(flat_idx, bucket_embedding)
    return out.reshape(*shape, DIM)


# speedup vs baseline: 8.5359x; 8.5359x over previous
import functools

import jax
import jax.numpy as jnp
from jax import lax
from jax.experimental import pallas as pl
from jax.experimental.pallas import tpu as pltpu
from jax.experimental.pallas import tpu_sc as plsc

NUM_ROWS = 3
DIM = 256

NUM_CORES = 2
NUM_SUBCORES = 16
NUM_WORKERS = NUM_CORES * NUM_SUBCORES
LANES = 16
SLOTS = DIM // LANES

N_TOTAL = 16 * 128 * 128
RPW = N_TOTAL // NUM_WORKERS
CHUNK = 128
NCHUNKS = RPW // CHUNK
NPAIRS = NCHUNKS // 2


@functools.partial(
    pl.kernel,
    out_type=jax.ShapeDtypeStruct((N_TOTAL * DIM,), jnp.float32),
    mesh=plsc.VectorSubcoreMesh(core_axis_name="c", subcore_axis_name="s"),
    compiler_params=pltpu.CompilerParams(needs_layout_passes=False),
    scratch_types=[
        pltpu.VMEM((RPW,), jnp.int32),
        pltpu.VMEM((NUM_ROWS * DIM,), jnp.float32),
        pltpu.VMEM((CHUNK * DIM,), jnp.float32),
        pltpu.VMEM((CHUNK * DIM,), jnp.float32),
        pltpu.SemaphoreType.DMA,
    ],
)
def _sc_expand(idx_hbm, table_hbm, out_hbm, idx_v, table_v, rows0, rows1, ssem):
    wid = lax.axis_index("s") * NUM_CORES + lax.axis_index("c")
    base = wid * RPW
    pltpu.sync_copy(table_hbm, table_v)
    pltpu.sync_copy(idx_hbm.at[pl.ds(base, RPW)], idx_v)

    def bucket_body(i, carry):
        s = idx_v[pl.ds(i * LANES, LANES)]
        idx_v[pl.ds(i * LANES, LANES)] = jnp.clip(s - 1, 0, NUM_ROWS - 1) * DIM
        return carry

    lax.fori_loop(0, RPW // LANES, bucket_body, 0)

    iota = lax.iota(jnp.int32, LANES)
    slot_off = [iota + (s * LANES) for s in range(SLOTS)]

    def expand_chunk(c, buf):
        def row_body(r, carry):
            bvec = plsc.load_gather(idx_v, [jnp.full((LANES,), r, jnp.int32)])
            for s in range(SLOTS):
                vals = plsc.load_gather(table_v, [bvec + slot_off[s]])
                rbase = (r - c * CHUNK) * DIM
                buf[pl.ds(rbase + s * LANES, LANES)] = vals
            return carry

        lax.fori_loop(c * CHUNK, (c + 1) * CHUNK, row_body, 0)

    def start_scatter(c, buf):
        return pltpu.async_copy(
            buf, out_hbm.at[pl.ds((base + c * CHUNK) * DIM, CHUNK * DIM)], ssem
        )

    def drain_one(buf):
        pltpu.make_async_copy(
            buf, out_hbm.at[pl.ds(base * DIM, CHUNK * DIM)], ssem
        ).wait()

    def pair_body(g, carry):
        for par, buf in ((0, rows0), (1, rows1)):
            c = g * 2 + par

            @pl.when(g > 0)
            def _():
                drain_one(buf)

            expand_chunk(c, buf)
            start_scatter(c, buf)
        return carry

    lax.fori_loop(0, NPAIRS, pair_body, 0)
    drain_one(rows0)
    drain_one(rows1)


def kernel(path_length, bucket_embedding):
    shape = path_length.shape
    flat_idx = path_length.reshape(-1).astype(jnp.int32)
    out = _sc_expand(flat_idx, bucket_embedding.reshape(-1))
    return out.reshape(*shape, DIM)


# speedup vs baseline: 19.3652x; 2.2687x over previous
import functools

import jax
import jax.numpy as jnp
from jax import lax
from jax.experimental import pallas as pl
from jax.experimental.pallas import tpu as pltpu
from jax.experimental.pallas import tpu_sc as plsc

NUM_ROWS = 3
DIM = 256

NUM_CORES = 2
NUM_SUBCORES = 16
NUM_WORKERS = NUM_CORES * NUM_SUBCORES
LANES = 16
SLOTS = DIM // LANES

N_TOTAL = 16 * 128 * 128
RPW = N_TOTAL // NUM_WORKERS
CHUNK = 128
NCHUNKS = RPW // CHUNK
NPAIRS = NCHUNKS // 2


@functools.partial(
    pl.kernel,
    out_type=jax.ShapeDtypeStruct((N_TOTAL * DIM,), jnp.float32),
    mesh=plsc.VectorSubcoreMesh(core_axis_name="c", subcore_axis_name="s"),
    compiler_params=pltpu.CompilerParams(needs_layout_passes=False),
    scratch_types=[
        pltpu.VMEM((RPW,), jnp.int32),
        pltpu.VMEM((NUM_ROWS * DIM,), jnp.float32),
        pltpu.VMEM((CHUNK * DIM,), jnp.float32),
        pltpu.VMEM((CHUNK * DIM,), jnp.float32),
        pltpu.SemaphoreType.DMA,
    ],
)
def _sc_expand(idx_hbm, table_hbm, out_hbm, idx_v, table_v, rows0, rows1, ssem):
    wid = lax.axis_index("s") * NUM_CORES + lax.axis_index("c")
    base = wid * RPW
    pltpu.sync_copy(table_hbm, table_v)
    pltpu.sync_copy(idx_hbm.at[pl.ds(base, RPW)], idx_v)

    def bucket_body(i, carry):
        s = idx_v[pl.ds(i * LANES, LANES)]
        idx_v[pl.ds(i * LANES, LANES)] = jnp.clip(s - 1, 0, NUM_ROWS - 1) * DIM
        return carry

    lax.fori_loop(0, RPW // LANES, bucket_body, 0)

    iota = lax.iota(jnp.int32, LANES)
    slot_off = [iota + (s * LANES) for s in range(SLOTS)]

    def expand_chunk(c, buf):
        @plsc.parallel_loop(c * CHUNK, (c + 1) * CHUNK, step=1, unroll=4)
        def row_body(r):
            bvec = plsc.load_gather(idx_v, [jnp.full((LANES,), r, jnp.int32)])
            rbase = (r - c * CHUNK) * DIM
            for s in range(SLOTS):
                vals = plsc.load_gather(table_v, [bvec + slot_off[s]])
                buf[pl.ds(rbase + s * LANES, LANES)] = vals

    def start_scatter(c, buf):
        return pltpu.async_copy(
            buf, out_hbm.at[pl.ds((base + c * CHUNK) * DIM, CHUNK * DIM)], ssem
        )

    def drain_one(buf):
        pltpu.make_async_copy(
            buf, out_hbm.at[pl.ds(base * DIM, CHUNK * DIM)], ssem
        ).wait()

    def pair_body(g, carry):
        for par, buf in ((0, rows0), (1, rows1)):
            c = g * 2 + par

            @pl.when(g > 0)
            def _():
                drain_one(buf)

            expand_chunk(c, buf)
            start_scatter(c, buf)
        return carry

    lax.fori_loop(0, NPAIRS, pair_body, 0)
    drain_one(rows0)
    drain_one(rows1)


def kernel(path_length, bucket_embedding):
    shape = path_length.shape
    flat_idx = path_length.reshape(-1).astype(jnp.int32)
    out = _sc_expand(flat_idx, bucket_embedding.reshape(-1))
    return out.reshape(*shape, DIM)
